# baseline (speedup 1.0000x reference)
import functools

import jax
import jax.numpy as jnp
from jax import lax
from jax.experimental import pallas as pl
from jax.experimental.pallas import tpu as pltpu
from jax.experimental.pallas import tpu_sc as plsc

N_USERS = 100000
N_ITEMS = 100000
K = 128
B = 16384
L = 16


@functools.lru_cache(maxsize=None)
def _build_sc_kernel():
    info = plsc.get_sparse_core_info()
    NC, NS = info.num_cores, info.num_subcores
    NW = NC * NS
    BPW = B // NW
    C = 64
    NCHUNK = BPW // C
    NGROUP = C // L

    mesh = plsc.VectorSubcoreMesh(core_axis_name="c", subcore_axis_name="s")

    @functools.partial(
        pl.kernel,
        out_type=(
            jax.ShapeDtypeStruct((B,), jnp.float32),
            jax.ShapeDtypeStruct((B, K), jnp.float32),
            jax.ShapeDtypeStruct((B, K), jnp.float32),
            jax.ShapeDtypeStruct((B,), jnp.float32),
        ),
        mesh=mesh,
        compiler_params=pltpu.CompilerParams(needs_layout_passes=False),
        scratch_types=[
            pltpu.VMEM((BPW,), jnp.int32),
            pltpu.VMEM((BPW,), jnp.int32),
            pltpu.VMEM((BPW,), jnp.float32),
            pltpu.VMEM((BPW,), jnp.float32),
            pltpu.VMEM((2, C, K), jnp.float32),
            pltpu.VMEM((2, C, K), jnp.float32),
            pltpu.VMEM((2, C, K), jnp.float32),
            pltpu.VMEM((L * 17,), jnp.float32),
            pltpu.SemaphoreType.DMA,
            pltpu.SemaphoreType.DMA,
            pltpu.SemaphoreType.DMA,
            pltpu.SemaphoreType.DMA,
            pltpu.SemaphoreType.DMA,
            pltpu.SemaphoreType.DMA,
            pltpu.SemaphoreType.DMA,
        ],
    )
    def emb_icd(uidx_hbm, iidx_hbm, know_hbm, theta_hbm, a_hbm, b_hbm,
                pred_out, theta_out, a_out, b_out,
                idx_u, idx_i, b_buf, pred_buf,
                theta_buf, a_buf, know_buf, tile_t,
                sem_t, sem_a, sem_k, semw_t, semw_a, sem_b, sem_wb):
        wid = lax.axis_index("s") * NC + lax.axis_index("c")
        base = wid * BPW
        lane = lax.iota(jnp.int32, L)
        lane17 = lane * 17

        pltpu.sync_copy(uidx_hbm.at[pl.ds(base, BPW)], idx_u)
        pltpu.sync_copy(iidx_hbm.at[pl.ds(base, BPW)], idx_i)
        b_cp = pltpu.async_copy(b_hbm.at[idx_i], b_buf, sem_b)

        def issue_gathers(c, s):
            rb = base + c * C
            pltpu.async_copy(
                theta_hbm.at[idx_u.at[pl.ds(c * C, C)]], theta_buf.at[s],
                sem_t)
            pltpu.async_copy(
                a_hbm.at[idx_i.at[pl.ds(c * C, C)]], a_buf.at[s], sem_a)
            pltpu.async_copy(
                know_hbm.at[pl.ds(rb, C)], know_buf.at[s], sem_k)

        issue_gathers(0, 0)
        b_cp.wait()
        b_wb = pltpu.async_copy(b_buf, b_out.at[pl.ds(base, BPW)], sem_wb)

        def chunk(c, _):
            s = c & 1
            rb = base + c * C
            pltpu.make_async_copy(theta_hbm.at[pl.ds(0, C)],
                                  theta_buf.at[s], sem_t).wait()
            pltpu.make_async_copy(a_hbm.at[pl.ds(0, C)],
                                  a_buf.at[s], sem_a).wait()
            pltpu.make_async_copy(know_hbm.at[pl.ds(0, C)],
                                  know_buf.at[s], sem_k).wait()

            @pl.when(c > 0)
            def _():
                pltpu.make_async_copy(theta_buf.at[s], theta_out.at[
                    pl.ds(0, C)], semw_t).wait()
                pltpu.make_async_copy(a_buf.at[s], a_out.at[
                    pl.ds(0, C)], semw_a).wait()

            pltpu.async_copy(theta_buf.at[s], theta_out.at[pl.ds(rb, C)],
                             semw_t)
            pltpu.async_copy(a_buf.at[s], a_out.at[pl.ds(rb, C)], semw_a)

            @pl.when(c + 1 < NCHUNK)
            def _():
                issue_gathers(c + 1, 1 - s)

            def group(g, _):
                def rowblk(i, _):
                    for k4 in range(4):
                        r16 = i * 4 + k4
                        r = g * L + r16
                        sl0 = pl.ds(0, L)
                        acc = (theta_buf[s, r, sl0] * a_buf[s, r, sl0]
                               * know_buf[s, r, sl0])
                        for j in range(1, K // L):
                            sl = pl.ds(j * L, L)
                            acc = acc + (theta_buf[s, r, sl]
                                         * a_buf[s, r, sl]
                                         * know_buf[s, r, sl])
                        plsc.store_scatter(tile_t, [lane17 + r16], acc)
                    return 0

                lax.fori_loop(0, L // 4, rowblk, 0, unroll=False)
                ssum = tile_t[pl.ds(0, L)]
                for l in range(1, L):
                    ssum = ssum + tile_t[pl.ds(l * 17, L)]
                off = c * C + g * L
                x = ssum - b_buf[pl.ds(off, L)]
                pred_buf[pl.ds(off, L)] = 1.0 / (1.0 + jnp.exp(-x))
                return 0

            lax.fori_loop(0, NGROUP, group, 0, unroll=False)
            return 0

        lax.fori_loop(0, NCHUNK, chunk, 0, unroll=False)

        s_last = (NCHUNK - 1) & 1
        pltpu.make_async_copy(theta_buf.at[s_last],
                              theta_out.at[pl.ds(0, C)], semw_t).wait()
        pltpu.make_async_copy(a_buf.at[s_last],
                              a_out.at[pl.ds(0, C)], semw_a).wait()
        b_wb.wait()
        pltpu.sync_copy(pred_buf, pred_out.at[pl.ds(base, BPW)])

    return emb_icd


@jax.jit
def kernel(user_idx, item_idx, know, theta_table, a_table, b_table):
    emb_icd = _build_sc_kernel()
    pred, theta, a, b_flat = emb_icd(
        user_idx.astype(jnp.int32),
        item_idx.astype(jnp.int32),
        know,
        theta_table,
        a_table,
        b_table.reshape(-1),
    )
    return pred, theta, a, b_flat[:, None]


# speedup vs baseline: 1.0090x; 1.0090x over previous
import functools

import jax
import jax.numpy as jnp
from jax import lax
from jax.experimental import pallas as pl
from jax.experimental.pallas import tpu as pltpu
from jax.experimental.pallas import tpu_sc as plsc

N_USERS = 100000
N_ITEMS = 100000
K = 128
B = 16384
L = 16


@functools.lru_cache(maxsize=None)
def _build_sc_kernel():
    info = plsc.get_sparse_core_info()
    NC, NS = info.num_cores, info.num_subcores
    NW = NC * NS
    BPW = B // NW
    C = 128
    NCHUNK = BPW // C
    NGROUP = C // L

    mesh = plsc.VectorSubcoreMesh(core_axis_name="c", subcore_axis_name="s")

    @functools.partial(
        pl.kernel,
        out_type=(
            jax.ShapeDtypeStruct((B,), jnp.float32),
            jax.ShapeDtypeStruct((B, K), jnp.float32),
            jax.ShapeDtypeStruct((B, K), jnp.float32),
            jax.ShapeDtypeStruct((B,), jnp.float32),
        ),
        mesh=mesh,
        compiler_params=pltpu.CompilerParams(needs_layout_passes=False),
        scratch_types=[
            pltpu.VMEM((BPW,), jnp.int32),
            pltpu.VMEM((BPW,), jnp.int32),
            pltpu.VMEM((BPW,), jnp.float32),
            pltpu.VMEM((BPW,), jnp.float32),
            pltpu.VMEM((2, C, K), jnp.float32),
            pltpu.VMEM((2, C, K), jnp.float32),
            pltpu.VMEM((2, C, K), jnp.float32),
            pltpu.VMEM((L * 17,), jnp.float32),
            pltpu.SemaphoreType.DMA,
            pltpu.SemaphoreType.DMA,
            pltpu.SemaphoreType.DMA,
            pltpu.SemaphoreType.DMA,
            pltpu.SemaphoreType.DMA,
            pltpu.SemaphoreType.DMA,
            pltpu.SemaphoreType.DMA,
        ],
    )
    def emb_icd(uidx_hbm, iidx_hbm, know_hbm, theta_hbm, a_hbm, b_hbm,
                pred_out, theta_out, a_out, b_out,
                idx_u, idx_i, b_buf, pred_buf,
                theta_buf, a_buf, know_buf, tile_t,
                sem_t, sem_a, sem_k, semw_t, semw_a, sem_b, sem_wb):
        wid = lax.axis_index("s") * NC + lax.axis_index("c")
        base = wid * BPW
        lane = lax.iota(jnp.int32, L)
        lane17 = lane * 17

        u_cp = pltpu.async_copy(uidx_hbm.at[pl.ds(base, BPW)], idx_u, sem_t)
        i_cp = pltpu.async_copy(iidx_hbm.at[pl.ds(base, BPW)], idx_i, sem_a)
        u_cp.wait()
        i_cp.wait()
        b_cp = pltpu.async_copy(b_hbm.at[idx_i], b_buf, sem_b)

        def issue_gathers(c, s):
            rb = base + c * C
            pltpu.async_copy(
                theta_hbm.at[idx_u.at[pl.ds(c * C, C)]], theta_buf.at[s],
                sem_t)
            pltpu.async_copy(
                a_hbm.at[idx_i.at[pl.ds(c * C, C)]], a_buf.at[s], sem_a)
            pltpu.async_copy(
                know_hbm.at[pl.ds(rb, C)], know_buf.at[s], sem_k)

        issue_gathers(0, 0)
        b_cp.wait()
        b_wb = pltpu.async_copy(b_buf, b_out.at[pl.ds(base, BPW)], sem_wb)

        def chunk(c, _):
            s = c & 1
            rb = base + c * C
            pltpu.make_async_copy(theta_hbm.at[pl.ds(0, C)],
                                  theta_buf.at[s], sem_t).wait()
            pltpu.make_async_copy(a_hbm.at[pl.ds(0, C)],
                                  a_buf.at[s], sem_a).wait()
            pltpu.make_async_copy(know_hbm.at[pl.ds(0, C)],
                                  know_buf.at[s], sem_k).wait()

            @pl.when(c > 0)
            def _():
                pltpu.make_async_copy(theta_buf.at[s], theta_out.at[
                    pl.ds(0, C)], semw_t).wait()
                pltpu.make_async_copy(a_buf.at[s], a_out.at[
                    pl.ds(0, C)], semw_a).wait()

            pltpu.async_copy(theta_buf.at[s], theta_out.at[pl.ds(rb, C)],
                             semw_t)
            pltpu.async_copy(a_buf.at[s], a_out.at[pl.ds(rb, C)], semw_a)

            @pl.when(c + 1 < NCHUNK)
            def _():
                issue_gathers(c + 1, 1 - s)

            def group(g, _):
                def rowblk(i, _):
                    for k4 in range(4):
                        r16 = i * 4 + k4
                        r = g * L + r16
                        sl0 = pl.ds(0, L)
                        acc = (theta_buf[s, r, sl0] * a_buf[s, r, sl0]
                               * know_buf[s, r, sl0])
                        for j in range(1, K // L):
                            sl = pl.ds(j * L, L)
                            acc = acc + (theta_buf[s, r, sl]
                                         * a_buf[s, r, sl]
                                         * know_buf[s, r, sl])
                        plsc.store_scatter(tile_t, [lane17 + r16], acc)
                    return 0

                lax.fori_loop(0, L // 4, rowblk, 0, unroll=False)
                ssum = tile_t[pl.ds(0, L)]
                for l in range(1, L):
                    ssum = ssum + tile_t[pl.ds(l * 17, L)]
                off = c * C + g * L
                x = ssum - b_buf[pl.ds(off, L)]
                pred_buf[pl.ds(off, L)] = 1.0 / (1.0 + jnp.exp(-x))
                return 0

            lax.fori_loop(0, NGROUP, group, 0, unroll=False)
            return 0

        lax.fori_loop(0, NCHUNK, chunk, 0, unroll=False)

        pltpu.sync_copy(pred_buf, pred_out.at[pl.ds(base, BPW)])
        s_last = (NCHUNK - 1) & 1
        pltpu.make_async_copy(theta_buf.at[s_last],
                              theta_out.at[pl.ds(0, C)], semw_t).wait()
        pltpu.make_async_copy(a_buf.at[s_last],
                              a_out.at[pl.ds(0, C)], semw_a).wait()
        b_wb.wait()

    return emb_icd


@jax.jit
def kernel(user_idx, item_idx, know, theta_table, a_table, b_table):
    emb_icd = _build_sc_kernel()
    pred, theta, a, b_flat = emb_icd(
        user_idx.astype(jnp.int32),
        item_idx.astype(jnp.int32),
        know,
        theta_table,
        a_table,
        b_table.reshape(-1),
    )
    return pred, theta, a, b_flat[:, None]


# speedup vs baseline: 1.0168x; 1.0077x over previous
import functools

import jax
import jax.numpy as jnp
from jax import lax
from jax.experimental import pallas as pl
from jax.experimental.pallas import tpu as pltpu
from jax.experimental.pallas import tpu_sc as plsc

N_USERS = 100000
N_ITEMS = 100000
K = 128
B = 16384
L = 16


@functools.lru_cache(maxsize=None)
def _build_sc_kernel():
    info = plsc.get_sparse_core_info()
    NC, NS = info.num_cores, info.num_subcores
    NW = NC * NS
    BPW = B // NW
    C = 128
    NCHUNK = BPW // C
    NGROUP = C // L

    mesh = plsc.VectorSubcoreMesh(core_axis_name="c", subcore_axis_name="s")

    @functools.partial(
        pl.kernel,
        out_type=(
            jax.ShapeDtypeStruct((B,), jnp.float32),
            jax.ShapeDtypeStruct((B, K), jnp.float32),
            jax.ShapeDtypeStruct((B, K), jnp.float32),
            jax.ShapeDtypeStruct((B,), jnp.float32),
        ),
        mesh=mesh,
        compiler_params=pltpu.CompilerParams(needs_layout_passes=False),
        scratch_types=[
            pltpu.VMEM((BPW,), jnp.int32),
            pltpu.VMEM((BPW,), jnp.int32),
            pltpu.VMEM((BPW,), jnp.float32),
            pltpu.VMEM((BPW,), jnp.float32),
            pltpu.VMEM((2, C, K), jnp.float32),
            pltpu.VMEM((2, C, K), jnp.float32),
            pltpu.VMEM((2, C, K), jnp.float32),
            pltpu.VMEM((L * 17,), jnp.float32),
            pltpu.SemaphoreType.DMA,
            pltpu.SemaphoreType.DMA,
            pltpu.SemaphoreType.DMA,
            pltpu.SemaphoreType.DMA,
            pltpu.SemaphoreType.DMA,
            pltpu.SemaphoreType.DMA,
            pltpu.SemaphoreType.DMA,
        ],
    )
    def emb_icd(uidx_hbm, iidx_hbm, know_hbm, theta_hbm, a_hbm, b_hbm,
                pred_out, theta_out, a_out, b_out,
                idx_u, idx_i, b_buf, pred_buf,
                theta_buf, a_buf, know_buf, tile_t,
                sem_t, sem_a, sem_k, semw_t, semw_a, sem_b, sem_wb):
        wid = lax.axis_index("s") * NC + lax.axis_index("c")
        base = wid * BPW
        lane = lax.iota(jnp.int32, L)
        lane17 = lane * 17

        u_cp = pltpu.async_copy(uidx_hbm.at[pl.ds(base, BPW)], idx_u, sem_t)
        i_cp = pltpu.async_copy(iidx_hbm.at[pl.ds(base, BPW)], idx_i, sem_a)
        u_cp.wait()
        i_cp.wait()

        def issue_gathers(c, s):
            rb = base + c * C
            pltpu.async_copy(
                theta_hbm.at[idx_u.at[pl.ds(c * C, C)]], theta_buf.at[s],
                sem_t)
            pltpu.async_copy(
                a_hbm.at[idx_i.at[pl.ds(c * C, C)]], a_buf.at[s], sem_a)
            pltpu.async_copy(
                know_hbm.at[pl.ds(rb, C)], know_buf.at[s], sem_k)

        issue_gathers(0, 0)
        b_cp = pltpu.async_copy(b_hbm.at[idx_i], b_buf, sem_b)
        b_cp.wait()
        b_wb = pltpu.async_copy(b_buf, b_out.at[pl.ds(base, BPW)], sem_wb)

        def chunk(c, _):
            s = c & 1
            rb = base + c * C
            @pl.when(c > 0)
            def _():
                pltpu.make_async_copy(theta_buf.at[s], theta_out.at[
                    pl.ds(0, C)], semw_t).wait()
                pltpu.make_async_copy(a_buf.at[s], a_out.at[
                    pl.ds(0, C)], semw_a).wait()

            pltpu.make_async_copy(theta_hbm.at[pl.ds(0, C)],
                                  theta_buf.at[s], sem_t).wait()
            pltpu.make_async_copy(a_hbm.at[pl.ds(0, C)],
                                  a_buf.at[s], sem_a).wait()
            pltpu.make_async_copy(know_hbm.at[pl.ds(0, C)],
                                  know_buf.at[s], sem_k).wait()

            @pl.when(c + 1 < NCHUNK)
            def _():
                issue_gathers(c + 1, 1 - s)

            pltpu.async_copy(theta_buf.at[s], theta_out.at[pl.ds(rb, C)],
                             semw_t)
            pltpu.async_copy(a_buf.at[s], a_out.at[pl.ds(rb, C)], semw_a)

            def group(g, _):
                def rowblk(i, _):
                    for k4 in range(4):
                        r16 = i * 4 + k4
                        r = g * L + r16
                        sl0 = pl.ds(0, L)
                        acc = (theta_buf[s, r, sl0] * a_buf[s, r, sl0]
                               * know_buf[s, r, sl0])
                        for j in range(1, K // L):
                            sl = pl.ds(j * L, L)
                            acc = acc + (theta_buf[s, r, sl]
                                         * a_buf[s, r, sl]
                                         * know_buf[s, r, sl])
                        plsc.store_scatter(tile_t, [lane17 + r16], acc)
                    return 0

                lax.fori_loop(0, L // 4, rowblk, 0, unroll=False)
                ssum = tile_t[pl.ds(0, L)]
                for l in range(1, L):
                    ssum = ssum + tile_t[pl.ds(l * 17, L)]
                off = c * C + g * L
                x = ssum - b_buf[pl.ds(off, L)]
                pred_buf[pl.ds(off, L)] = 1.0 / (1.0 + jnp.exp(-x))
                return 0

            lax.fori_loop(0, NGROUP, group, 0, unroll=False)
            return 0

        lax.fori_loop(0, NCHUNK, chunk, 0, unroll=False)

        pltpu.sync_copy(pred_buf, pred_out.at[pl.ds(base, BPW)])
        s_last = (NCHUNK - 1) & 1
        pltpu.make_async_copy(theta_buf.at[s_last],
                              theta_out.at[pl.ds(0, C)], semw_t).wait()
        pltpu.make_async_copy(a_buf.at[s_last],
                              a_out.at[pl.ds(0, C)], semw_a).wait()
        b_wb.wait()

    return emb_icd


@jax.jit
def kernel(user_idx, item_idx, know, theta_table, a_table, b_table):
    emb_icd = _build_sc_kernel()
    pred, theta, a, b_flat = emb_icd(
        user_idx.astype(jnp.int32),
        item_idx.astype(jnp.int32),
        know,
        theta_table,
        a_table,
        b_table.reshape(-1),
    )
    return pred, theta, a, b_flat[:, None]


# speedup vs baseline: 1.0252x; 1.0084x over previous
import functools

import jax
import jax.numpy as jnp
from jax import lax
from jax.experimental import pallas as pl
from jax.experimental.pallas import tpu as pltpu
from jax.experimental.pallas import tpu_sc as plsc

N_USERS = 100000
N_ITEMS = 100000
K = 128
B = 16384
L = 16


@functools.lru_cache(maxsize=None)
def _build_sc_kernel():
    info = plsc.get_sparse_core_info()
    NC, NS = info.num_cores, info.num_subcores
    NW = NC * NS
    BPW = B // NW
    C = 128
    NCHUNK = BPW // C
    NGROUP = C // L

    mesh = plsc.VectorSubcoreMesh(core_axis_name="c", subcore_axis_name="s")

    @functools.partial(
        pl.kernel,
        out_type=(
            jax.ShapeDtypeStruct((B,), jnp.float32),
            jax.ShapeDtypeStruct((B, K), jnp.float32),
            jax.ShapeDtypeStruct((B, K), jnp.float32),
            jax.ShapeDtypeStruct((B,), jnp.float32),
        ),
        mesh=mesh,
        compiler_params=pltpu.CompilerParams(needs_layout_passes=False),
        scratch_types=[
            pltpu.VMEM((BPW,), jnp.int32),
            pltpu.VMEM((BPW,), jnp.int32),
            pltpu.VMEM((BPW,), jnp.float32),
            pltpu.VMEM((BPW,), jnp.float32),
            pltpu.VMEM((2, C, K), jnp.float32),
            pltpu.VMEM((2, C, K), jnp.float32),
            pltpu.VMEM((2, C, K), jnp.float32),
            pltpu.VMEM((L * 17,), jnp.float32),
            pltpu.SemaphoreType.DMA,
            pltpu.SemaphoreType.DMA,
            pltpu.SemaphoreType.DMA,
            pltpu.SemaphoreType.DMA,
            pltpu.SemaphoreType.DMA,
            pltpu.SemaphoreType.DMA,
            pltpu.SemaphoreType.DMA,
        ],
    )
    def emb_icd(uidx_hbm, iidx_hbm, know_hbm, theta_hbm, a_hbm, b_hbm,
                pred_out, theta_out, a_out, b_out,
                idx_u, idx_i, b_buf, pred_buf,
                theta_buf, a_buf, know_buf, tile_t,
                sem_t, sem_a, sem_k, semw_t, semw_a, sem_b, sem_wb):
        wid = lax.axis_index("s") * NC + lax.axis_index("c")
        base = wid * BPW
        lane = lax.iota(jnp.int32, L)
        lane17 = lane * 17

        u_cp = pltpu.async_copy(uidx_hbm.at[pl.ds(base, BPW)], idx_u, sem_t)
        i_cp = pltpu.async_copy(iidx_hbm.at[pl.ds(base, BPW)], idx_i, sem_a)
        u_cp.wait()
        i_cp.wait()

        def issue_gathers(c, s):
            rb = base + c * C
            pltpu.async_copy(
                theta_hbm.at[idx_u.at[pl.ds(c * C, C)]], theta_buf.at[s],
                sem_t)
            pltpu.async_copy(
                a_hbm.at[idx_i.at[pl.ds(c * C, C)]], a_buf.at[s], sem_a)
            pltpu.async_copy(
                know_hbm.at[pl.ds(rb, C)], know_buf.at[s], sem_k)

        issue_gathers(0, 0)
        pltpu.async_copy(b_hbm.at[idx_i], b_buf, sem_b)

        def chunk(c, _):
            s = c & 1
            rb = base + c * C
            @pl.when(c > 0)
            def _():
                pltpu.make_async_copy(theta_buf.at[s], theta_out.at[
                    pl.ds(0, C)], semw_t).wait()
                pltpu.make_async_copy(a_buf.at[s], a_out.at[
                    pl.ds(0, C)], semw_a).wait()

            pltpu.make_async_copy(theta_hbm.at[pl.ds(0, C)],
                                  theta_buf.at[s], sem_t).wait()
            pltpu.make_async_copy(a_hbm.at[pl.ds(0, C)],
                                  a_buf.at[s], sem_a).wait()
            pltpu.make_async_copy(know_hbm.at[pl.ds(0, C)],
                                  know_buf.at[s], sem_k).wait()

            @pl.when(c + 1 < NCHUNK)
            def _():
                issue_gathers(c + 1, 1 - s)

            pltpu.async_copy(theta_buf.at[s], theta_out.at[pl.ds(rb, C)],
                             semw_t)
            pltpu.async_copy(a_buf.at[s], a_out.at[pl.ds(rb, C)], semw_a)

            @pl.when(c == 0)
            def _():
                pltpu.make_async_copy(b_hbm.at[pl.ds(0, BPW)], b_buf,
                                      sem_b).wait()
                pltpu.async_copy(b_buf, b_out.at[pl.ds(base, BPW)], sem_wb)

            def group(g, _):
                def rowblk(i, _):
                    for k4 in range(4):
                        r16 = i * 4 + k4
                        r = g * L + r16
                        sl0 = pl.ds(0, L)
                        acc = (theta_buf[s, r, sl0] * a_buf[s, r, sl0]
                               * know_buf[s, r, sl0])
                        for j in range(1, K // L):
                            sl = pl.ds(j * L, L)
                            acc = acc + (theta_buf[s, r, sl]
                                         * a_buf[s, r, sl]
                                         * know_buf[s, r, sl])
                        plsc.store_scatter(tile_t, [lane17 + r16], acc)
                    return 0

                lax.fori_loop(0, L // 4, rowblk, 0, unroll=False)
                ssum = tile_t[pl.ds(0, L)]
                for l in range(1, L):
                    ssum = ssum + tile_t[pl.ds(l * 17, L)]
                off = c * C + g * L
                x = ssum - b_buf[pl.ds(off, L)]
                pred_buf[pl.ds(off, L)] = 1.0 / (1.0 + jnp.exp(-x))
                return 0

            lax.fori_loop(0, NGROUP, group, 0, unroll=False)
            return 0

        lax.fori_loop(0, NCHUNK, chunk, 0, unroll=False)

        pltpu.sync_copy(pred_buf, pred_out.at[pl.ds(base, BPW)])
        s_last = (NCHUNK - 1) & 1
        pltpu.make_async_copy(theta_buf.at[s_last],
                              theta_out.at[pl.ds(0, C)], semw_t).wait()
        pltpu.make_async_copy(a_buf.at[s_last],
                              a_out.at[pl.ds(0, C)], semw_a).wait()
        pltpu.make_async_copy(b_buf, b_out.at[pl.ds(base, BPW)],
                              sem_wb).wait()

    return emb_icd


@jax.jit
def kernel(user_idx, item_idx, know, theta_table, a_table, b_table):
    emb_icd = _build_sc_kernel()
    pred, theta, a, b_flat = emb_icd(
        user_idx.astype(jnp.int32),
        item_idx.astype(jnp.int32),
        know,
        theta_table,
        a_table,
        b_table.reshape(-1),
    )
    return pred, theta, a, b_flat[:, None]
